# baseline (speedup 1.0000x reference)
import jax
import jax.numpy as jnp
from jax.experimental import pallas as pl
from jax.experimental.pallas import tpu as pltpu


N_EXP = 8
TOP_K = 2


def _moe_body(x_ref, gw_ref, gb_ref, w_ref, b_ref, out_ref, xc_ref, c_ref):
    d = x_ref.shape[1]
    tm2 = out_ref.shape[0]
    mi = pl.program_id(1)

    @pl.when((pl.program_id(0) == 0) & (mi == 0))
    def _gate():
        xb = x_ref[...]
        logits = jax.lax.dot_general(
            xb, gw_ref[...], (((1,), (1,)), ((), ())),
            precision=jax.lax.Precision.DEFAULT,
            preferred_element_type=jnp.float32,
        ) + gb_ref[...]
        m = jnp.max(logits, axis=-1, keepdims=True)
        p = jnp.exp(logits - m)
        s = p / jnp.sum(p, axis=-1, keepdims=True)
        ii = jax.lax.broadcasted_iota(jnp.int32, s.shape, 1)
        m1 = jnp.max(s, axis=-1, keepdims=True)
        i1 = jnp.min(jnp.where(s == m1, ii, N_EXP), axis=-1, keepdims=True)
        s2 = jnp.where(ii == i1, -jnp.inf, s)
        m2 = jnp.max(s2, axis=-1, keepdims=True)
        i2 = jnp.min(jnp.where(s2 == m2, ii, N_EXP), axis=-1, keepdims=True)
        sel = (ii == i1) | (ii == i2)
        c = jnp.where(sel, s, 0.0) * (1.0 / TOP_K)
        c_ref[...] = c
        cb = c.astype(jnp.bfloat16)
        for e in range(N_EXP):
            xc_ref[:, e * d:(e + 1) * d] = xb * cb[:, e:e + 1]

    wb = w_ref[...].astype(jnp.bfloat16)

    @pl.when(mi == 0)
    def _half0():
        t = jnp.dot(xc_ref[:tm2, :], wb, preferred_element_type=jnp.float32)
        t += jnp.dot(c_ref[:tm2, :], b_ref[...], preferred_element_type=jnp.float32)
        out_ref[...] = t

    @pl.when(mi == 1)
    def _half1():
        t = jnp.dot(xc_ref[tm2:, :], wb, preferred_element_type=jnp.float32)
        t += jnp.dot(c_ref[tm2:, :], b_ref[...], preferred_element_type=jnp.float32)
        out_ref[...] = t


def kernel(sequences, expert_weights, expert_biases, gating_w, gating_b):
    n, p, d = sequences.shape
    tokens = n * p
    d_inner = expert_biases.shape[-1]
    x = sequences.reshape(tokens, d)
    k_all = N_EXP * d

    tn = 128
    n_m = 2
    tm2 = tokens // n_m
    n_tiles = d_inner // tn
    out = pl.pallas_call(
        _moe_body,
        grid=(n_tiles, n_m),
        in_specs=[
            pl.BlockSpec((tokens, d), lambda ni, mi: (0, 0)),
            pl.BlockSpec((N_EXP, d), lambda ni, mi: (0, 0)),
            pl.BlockSpec((1, N_EXP), lambda ni, mi: (0, 0)),
            pl.BlockSpec((k_all, tn), lambda ni, mi: (0, ni)),
            pl.BlockSpec((N_EXP, tn), lambda ni, mi: (0, ni)),
        ],
        out_specs=pl.BlockSpec((tm2, tn), lambda ni, mi: (mi, ni)),
        out_shape=jax.ShapeDtypeStruct((tokens, d_inner), jnp.float32),
        scratch_shapes=[
            pltpu.VMEM((tokens, k_all), jnp.bfloat16),
            pltpu.VMEM((tokens, N_EXP), jnp.float32),
        ],
    )(x.astype(jnp.bfloat16), gating_w.astype(jnp.bfloat16),
      gating_b.reshape(1, N_EXP), expert_weights.reshape(k_all, d_inner),
      expert_biases)

    return out.reshape(n, p, d_inner)


# speedup vs baseline: 1.9251x; 1.9251x over previous
import jax
import jax.numpy as jnp
from jax.experimental import pallas as pl
from jax.experimental.pallas import tpu as pltpu


N_EXP = 8
TOP_K = 2


def _moe_body(x_ref, gw_ref, gb_ref, w_ref, b_ref, out_ref, xc_ref, c_ref):
    d = x_ref.shape[1]

    @pl.when(pl.program_id(1) == 0)
    def _gate():
        x = x_ref[...]
        logits = jax.lax.dot_general(
            x, gw_ref[...], (((1,), (1,)), ((), ())),
            precision=jax.lax.Precision.DEFAULT,
            preferred_element_type=jnp.float32,
        ) + gb_ref[...]
        m = jnp.max(logits, axis=-1, keepdims=True)
        p = jnp.exp(logits - m)
        s = p / jnp.sum(p, axis=-1, keepdims=True)
        ii = jax.lax.broadcasted_iota(jnp.int32, s.shape, 1)
        m1 = jnp.max(s, axis=-1, keepdims=True)
        i1 = jnp.min(jnp.where(s == m1, ii, N_EXP), axis=-1, keepdims=True)
        s2 = jnp.where(ii == i1, -jnp.inf, s)
        m2 = jnp.max(s2, axis=-1, keepdims=True)
        i2 = jnp.min(jnp.where(s2 == m2, ii, N_EXP), axis=-1, keepdims=True)
        sel = (ii == i1) | (ii == i2)
        c = jnp.where(sel, s, 0.0) * (1.0 / TOP_K)
        c_ref[...] = c
        for e in range(N_EXP):
            xc_ref[:, e * d:(e + 1) * d] = (x * c[:, e:e + 1]).astype(jnp.bfloat16)

    t = jnp.dot(xc_ref[...], w_ref[...].astype(jnp.bfloat16),
                preferred_element_type=jnp.float32)
    t += jnp.dot(c_ref[...], b_ref[...], preferred_element_type=jnp.float32)
    out_ref[...] = t


def kernel(sequences, expert_weights, expert_biases, gating_w, gating_b):
    n, p, d = sequences.shape
    tokens = n * p
    d_inner = expert_biases.shape[-1]
    x = sequences.reshape(tokens, d)
    k_all = N_EXP * d

    tn = 256
    tm = tokens // 2
    n_tiles = d_inner // tn
    out = pl.pallas_call(
        _moe_body,
        grid=(2, n_tiles),
        in_specs=[
            pl.BlockSpec((tm, d), lambda mi, ni: (mi, 0)),
            pl.BlockSpec((N_EXP, d), lambda mi, ni: (0, 0)),
            pl.BlockSpec((1, N_EXP), lambda mi, ni: (0, 0)),
            pl.BlockSpec((k_all, tn), lambda mi, ni: (0, ni)),
            pl.BlockSpec((N_EXP, tn), lambda mi, ni: (0, ni)),
        ],
        out_specs=pl.BlockSpec((tm, tn), lambda mi, ni: (mi, ni)),
        out_shape=jax.ShapeDtypeStruct((tokens, d_inner), jnp.float32),
        scratch_shapes=[
            pltpu.VMEM((tm, k_all), jnp.bfloat16),
            pltpu.VMEM((tm, N_EXP), jnp.float32),
        ],
    )(x, gating_w, gating_b.reshape(1, N_EXP), expert_weights.reshape(k_all, d_inner), expert_biases)

    return out.reshape(n, p, d_inner)
